# Initial kernel scaffold; baseline (speedup 1.0000x reference)
import jax
import jax.numpy as jnp
from jax.experimental import pallas as pl


def kernel(x, weight, e_score_correction_bias):
    raise NotImplementedError("write your pallas kernel here")


# speedup vs baseline: 3.1500x; 3.1500x over previous
import functools

import jax
import jax.numpy as jnp
from jax.experimental import pallas as pl

N_TOK = 8192
DIM = 7168
N_EXPERTS = 256
N_GROUPS = 8
GROUP_SIZE = N_EXPERTS // N_GROUPS
TOPK_GROUPS = 4
TOPK = 8
ROUTE_SCALE = 2.5

NEG = -1e30


def _gate_kernel(x_ref, w_ref, b_ref, wout_ref, iout_ref):
    x = x_ref[...]
    w = w_ref[...]
    scores = jax.lax.dot_general(
        x, w, (((1,), (1,)), ((), ())), preferred_element_type=jnp.float32
    )
    sig = jax.nn.sigmoid(scores)
    s = sig + b_ref[...]

    br = s.shape[0]
    lane = jax.lax.broadcasted_iota(jnp.int32, (br, N_EXPERTS), 1)

    gscores = []
    for g in range(N_GROUPS):
        in_g = (lane >= g * GROUP_SIZE) & (lane < (g + 1) * GROUP_SIZE)
        sg = jnp.where(in_g, s, NEG)
        m1 = jnp.max(sg, axis=1, keepdims=True)
        idx1 = jnp.min(jnp.where(sg == m1, lane, N_EXPERTS), axis=1, keepdims=True)
        sg2 = jnp.where(lane == idx1, NEG, sg)
        m2 = jnp.max(sg2, axis=1, keepdims=True)
        gscores.append(m1 + m2)
    gs = jnp.concatenate(gscores, axis=1)

    giota = jax.lax.broadcasted_iota(jnp.int32, (br, N_GROUPS), 1)
    rank = jnp.zeros((br, N_GROUPS), dtype=jnp.int32)
    for h in range(N_GROUPS):
        gh = gs[:, h : h + 1]
        beats = (gh > gs) | ((gh == gs) & (h < giota))
        rank = rank + beats.astype(jnp.int32)
    sel = rank < TOPK_GROUPS

    keep = jnp.zeros((br, N_EXPERTS), dtype=jnp.bool_)
    for g in range(N_GROUPS):
        in_g = (lane >= g * GROUP_SIZE) & (lane < (g + 1) * GROUP_SIZE)
        keep = keep | (in_g & sel[:, g : g + 1])
    cur = jnp.where(keep, s, NEG)

    w_cols = []
    i_cols = []
    for _ in range(TOPK):
        m = jnp.max(cur, axis=1, keepdims=True)
        idx = jnp.min(jnp.where(cur == m, lane, N_EXPERTS), axis=1, keepdims=True)
        hit = lane == idx
        w_cols.append(jnp.sum(jnp.where(hit, sig, 0.0), axis=1, keepdims=True))
        i_cols.append(idx)
        cur = jnp.where(hit, NEG, cur)
    wsel = jnp.concatenate(w_cols, axis=1)
    isel = jnp.concatenate(i_cols, axis=1)

    wsel = wsel / jnp.sum(wsel, axis=1, keepdims=True) * ROUTE_SCALE
    wout_ref[...] = wsel
    iout_ref[...] = isel


@functools.partial(jax.jit, static_argnames=())
def kernel(x, weight, e_score_correction_bias):
    n = x.shape[0]
    br = 256
    bias2d = e_score_correction_bias.reshape(1, N_EXPERTS)
    wout, iout = pl.pallas_call(
        _gate_kernel,
        grid=(n // br,),
        in_specs=[
            pl.BlockSpec((br, DIM), lambda i: (i, 0)),
            pl.BlockSpec((N_EXPERTS, DIM), lambda i: (0, 0)),
            pl.BlockSpec((1, N_EXPERTS), lambda i: (0, 0)),
        ],
        out_specs=[
            pl.BlockSpec((br, TOPK), lambda i: (i, 0)),
            pl.BlockSpec((br, TOPK), lambda i: (i, 0)),
        ],
        out_shape=[
            jax.ShapeDtypeStruct((n, TOPK), jnp.float32),
            jax.ShapeDtypeStruct((n, TOPK), jnp.int32),
        ],
    )(x, weight, bias2d)
    return wout, iout


# speedup vs baseline: 3.5349x; 1.1222x over previous
import functools

import jax
import jax.numpy as jnp
from jax.experimental import pallas as pl

N_TOK = 8192
DIM = 7168
N_EXPERTS = 256
N_GROUPS = 8
GROUP_SIZE = N_EXPERTS // N_GROUPS
TOPK_GROUPS = 4
TOPK = 8
ROUTE_SCALE = 2.5

NEG = -1e30


def _gate_kernel(x_ref, w_ref, b_ref, wout_ref, iout_ref):
    x = x_ref[...]
    w = w_ref[...]
    scores = jax.lax.dot_general(
        x, w, (((1,), (1,)), ((), ())), preferred_element_type=jnp.float32
    )
    sig = jax.nn.sigmoid(scores)
    s = sig + b_ref[...]

    br = s.shape[0]
    lane = jax.lax.broadcasted_iota(jnp.int32, (br, N_EXPERTS), 1)

    m1 = s
    m2 = jnp.full_like(s, NEG)
    for d in (1, 2, 4, 8, 16):
        up = (lane & d) == 0
        pm1 = jnp.where(up, jnp.roll(m1, -d, axis=1), jnp.roll(m1, d, axis=1))
        pm2 = jnp.where(up, jnp.roll(m2, -d, axis=1), jnp.roll(m2, d, axis=1))
        hi = jnp.maximum(m1, pm1)
        lo = jnp.minimum(m1, pm1)
        m2 = jnp.maximum(lo, jnp.maximum(m2, pm2))
        m1 = hi
    gs = m1 + m2

    rank = jnp.zeros((br, N_EXPERTS), dtype=jnp.float32)
    for k in range(1, N_GROUPS):
        other = jnp.roll(gs, -k * GROUP_SIZE, axis=1)
        tie_wins = lane >= (N_GROUPS - k) * GROUP_SIZE
        beats = (other > gs) | ((other == gs) & tie_wins)
        rank = rank + beats.astype(jnp.float32)
    cur = jnp.where(rank < TOPK_GROUPS, s, NEG)

    w_cols = []
    i_cols = []
    for _ in range(TOPK):
        m = jnp.max(cur, axis=1, keepdims=True)
        idx = jnp.min(jnp.where(cur == m, lane, N_EXPERTS), axis=1, keepdims=True)
        hit = lane == idx
        w_cols.append(jnp.sum(jnp.where(hit, sig, 0.0), axis=1, keepdims=True))
        i_cols.append(idx)
        cur = jnp.where(hit, NEG, cur)
    wsel = jnp.concatenate(w_cols, axis=1)
    isel = jnp.concatenate(i_cols, axis=1)

    wsel = wsel / jnp.sum(wsel, axis=1, keepdims=True) * ROUTE_SCALE
    wout_ref[...] = wsel
    iout_ref[...] = isel


@functools.partial(jax.jit, static_argnames=())
def kernel(x, weight, e_score_correction_bias):
    n = x.shape[0]
    br = 256
    bias2d = e_score_correction_bias.reshape(1, N_EXPERTS)
    wout, iout = pl.pallas_call(
        _gate_kernel,
        grid=(n // br,),
        in_specs=[
            pl.BlockSpec((br, DIM), lambda i: (i, 0)),
            pl.BlockSpec((N_EXPERTS, DIM), lambda i: (0, 0)),
            pl.BlockSpec((1, N_EXPERTS), lambda i: (0, 0)),
        ],
        out_specs=[
            pl.BlockSpec((br, TOPK), lambda i: (i, 0)),
            pl.BlockSpec((br, TOPK), lambda i: (i, 0)),
        ],
        out_shape=[
            jax.ShapeDtypeStruct((n, TOPK), jnp.float32),
            jax.ShapeDtypeStruct((n, TOPK), jnp.int32),
        ],
    )(x, weight, bias2d)
    return wout, iout


# speedup vs baseline: 4.3962x; 1.2436x over previous
import functools

import jax
import jax.numpy as jnp
from jax.experimental import pallas as pl

N_TOK = 8192
DIM = 7168
N_EXPERTS = 256
N_GROUPS = 8
GROUP_SIZE = N_EXPERTS // N_GROUPS
TOPK_GROUPS = 4
TOPK = 8
ROUTE_SCALE = 2.5

NEG = -1e30


def _gate_kernel(x_ref, w_ref, b_ref, wout_ref, iout_ref):
    x = x_ref[...]
    w = w_ref[...]
    scores = jax.lax.dot_general(
        x, w, (((1,), (1,)), ((), ())), preferred_element_type=jnp.float32
    )
    sig = jax.nn.sigmoid(scores)
    s = sig + b_ref[...]

    br = s.shape[0]
    lane = jax.lax.broadcasted_iota(jnp.int32, (br, N_EXPERTS), 1)

    m1 = s
    m2 = jnp.full_like(s, NEG)
    for d in (1, 2, 4, 8, 16):
        up = (lane & d) == 0
        pm1 = jnp.where(up, jnp.roll(m1, -d, axis=1), jnp.roll(m1, d, axis=1))
        pm2 = jnp.where(up, jnp.roll(m2, -d, axis=1), jnp.roll(m2, d, axis=1))
        hi = jnp.maximum(m1, pm1)
        lo = jnp.minimum(m1, pm1)
        m2 = jnp.maximum(lo, jnp.maximum(m2, pm2))
        m1 = hi
    gs = m1 + m2

    rank = jnp.zeros((br, N_EXPERTS), dtype=jnp.float32)
    for k in range(1, N_GROUPS):
        other = jnp.roll(gs, -k * GROUP_SIZE, axis=1)
        tie_wins = lane >= (N_GROUPS - k) * GROUP_SIZE
        beats = (other > gs) | ((other == gs) & tie_wins)
        rank = rank + beats.astype(jnp.float32)
    cur = jnp.where(rank < TOPK_GROUPS, s, NEG)

    lanef = lane.astype(jnp.float32)
    w_cols = []
    i_cols = []
    for _ in range(TOPK):
        m = jnp.max(cur, axis=1, keepdims=True)
        idxf = jnp.min(jnp.where(cur == m, lanef, 1e9), axis=1, keepdims=True)
        hit = lanef == idxf
        w_cols.append(jnp.sum(jnp.where(hit, sig, 0.0), axis=1, keepdims=True))
        i_cols.append(idxf)
        cur = jnp.where(hit, NEG, cur)
    wsel = jnp.concatenate(w_cols, axis=1)
    isel = jnp.concatenate(i_cols, axis=1).astype(jnp.int32)

    wsel = wsel / jnp.sum(wsel, axis=1, keepdims=True) * ROUTE_SCALE
    wout_ref[...] = wsel
    iout_ref[...] = isel


@functools.partial(jax.jit, static_argnames=())
def kernel(x, weight, e_score_correction_bias):
    n = x.shape[0]
    br = 512
    bias2d = e_score_correction_bias.reshape(1, N_EXPERTS)
    wout, iout = pl.pallas_call(
        _gate_kernel,
        grid=(n // br,),
        in_specs=[
            pl.BlockSpec((br, DIM), lambda i: (i, 0)),
            pl.BlockSpec((N_EXPERTS, DIM), lambda i: (0, 0)),
            pl.BlockSpec((1, N_EXPERTS), lambda i: (0, 0)),
        ],
        out_specs=[
            pl.BlockSpec((br, TOPK), lambda i: (i, 0)),
            pl.BlockSpec((br, TOPK), lambda i: (i, 0)),
        ],
        out_shape=[
            jax.ShapeDtypeStruct((n, TOPK), jnp.float32),
            jax.ShapeDtypeStruct((n, TOPK), jnp.int32),
        ],
    )(x, weight, bias2d)
    return wout, iout


# speedup vs baseline: 5.3371x; 1.2140x over previous
import functools

import jax
import jax.numpy as jnp
from jax.experimental import pallas as pl

N_TOK = 8192
DIM = 7168
N_EXPERTS = 256
N_GROUPS = 8
GROUP_SIZE = N_EXPERTS // N_GROUPS
TOPK_GROUPS = 4
TOPK = 8
ROUTE_SCALE = 2.5

NEG = -1e30


def _gate_kernel(x_ref, w_ref, b_ref, bcast_ref, wout_ref, iout_ref):
    x = x_ref[...]
    w = w_ref[...]
    scores = jax.lax.dot_general(
        x, w, (((1,), (1,)), ((), ())), preferred_element_type=jnp.float32
    )
    sig = 0.5 * jnp.tanh(scores * 0.5) + 0.5
    s = sig + b_ref[...]

    br = s.shape[0]
    lane = jax.lax.broadcasted_iota(jnp.int32, (br, N_EXPERTS), 1)
    lmod = lane & (GROUP_SIZE - 1)

    m1 = s
    m2 = jnp.full_like(s, NEG)
    for d in (1, 2, 4, 8, 16):
        valid = lmod < (GROUP_SIZE - d)
        pm1 = jnp.where(valid, jnp.roll(m1, -d, axis=1), NEG)
        pm2 = jnp.where(valid, jnp.roll(m2, -d, axis=1), NEG)
        lo = jnp.minimum(m1, pm1)
        m1 = jnp.maximum(m1, pm1)
        m2 = jnp.maximum(lo, jnp.maximum(m2, pm2))
    gs = m1 + m2

    rank = jnp.zeros((br, N_EXPERTS), dtype=jnp.float32)
    for k in range(1, N_GROUPS):
        other = jnp.roll(gs, -k * GROUP_SIZE, axis=1)
        tie_wins = lane >= (N_GROUPS - k) * GROUP_SIZE
        beats = (other > gs) | ((other == gs) & tie_wins)
        rank = rank + beats.astype(jnp.float32)

    keep_sparse = jnp.where((rank < TOPK_GROUPS) & (lmod == 0), 1.0, 0.0)
    keep = jax.lax.dot_general(
        keep_sparse.astype(jnp.bfloat16), bcast_ref[...],
        (((1,), (0,)), ((), ())), preferred_element_type=jnp.float32,
    )
    cur = jnp.where(keep > 0.0, s, NEG)

    lanef = lane.astype(jnp.float32)
    w_cols = []
    i_cols = []
    for _ in range(TOPK):
        m = jnp.max(cur, axis=1, keepdims=True)
        idxf = jnp.min(jnp.where(cur == m, lanef, 1e9), axis=1, keepdims=True)
        hit = lanef == idxf
        w_cols.append(jnp.sum(jnp.where(hit, sig, 0.0), axis=1, keepdims=True))
        i_cols.append(idxf)
        cur = jnp.where(hit, NEG, cur)
    wsel = jnp.concatenate(w_cols, axis=1)
    isel = jnp.concatenate(i_cols, axis=1).astype(jnp.int32)

    wsel = wsel / jnp.sum(wsel, axis=1, keepdims=True) * ROUTE_SCALE
    wout_ref[...] = wsel
    iout_ref[...] = isel


@functools.partial(jax.jit, static_argnames=())
def kernel(x, weight, e_score_correction_bias):
    n = x.shape[0]
    br = 512
    bias2d = e_score_correction_bias.reshape(1, N_EXPERTS)
    jj = jax.lax.broadcasted_iota(jnp.int32, (N_EXPERTS, N_EXPERTS), 0)
    ee = jax.lax.broadcasted_iota(jnp.int32, (N_EXPERTS, N_EXPERTS), 1)
    bcast = ((jj == (ee // GROUP_SIZE) * GROUP_SIZE)).astype(jnp.bfloat16)
    wout, iout = pl.pallas_call(
        _gate_kernel,
        grid=(n // br,),
        in_specs=[
            pl.BlockSpec((br, DIM), lambda i: (i, 0)),
            pl.BlockSpec((N_EXPERTS, DIM), lambda i: (0, 0)),
            pl.BlockSpec((1, N_EXPERTS), lambda i: (0, 0)),
            pl.BlockSpec((N_EXPERTS, N_EXPERTS), lambda i: (0, 0)),
        ],
        out_specs=[
            pl.BlockSpec((br, TOPK), lambda i: (i, 0)),
            pl.BlockSpec((br, TOPK), lambda i: (i, 0)),
        ],
        out_shape=[
            jax.ShapeDtypeStruct((n, TOPK), jnp.float32),
            jax.ShapeDtypeStruct((n, TOPK), jnp.int32),
        ],
    )(x, weight, bias2d, bcast)
    return wout, iout
